# Initial kernel scaffold; baseline (speedup 1.0000x reference)
import jax
import jax.numpy as jnp
from jax.experimental import pallas as pl


def kernel(x, edge_index, Wl0, Wr0, b0, Wl1, Wr1, b1, Wl2, Wr2, b2):
    raise NotImplementedError("write your pallas kernel here")


# speedup vs baseline: 4.6911x; 4.6911x over previous
import functools

import jax
import jax.numpy as jnp
from jax import lax
from jax.experimental import pallas as pl
from jax.experimental.pallas import tpu as pltpu
from jax.experimental.pallas import tpu_sc as plsc

N = 10000
E = 320000
D = 128

NC = 2
NS = 16
NW = NC * NS
EPW = E // NW
C = 80
NCHUNK = EPW // C
SPT = 624
TAIL = N - NS * SPT
ZROWS = 16

_MESH = plsc.VectorSubcoreMesh(
    core_axis_name="c", subcore_axis_name="s", num_cores=NC, num_subcores=NS
)


def _worker_id():
    return lax.axis_index("c") * NS + lax.axis_index("s")


@functools.partial(
    pl.kernel,
    out_type=jax.ShapeDtypeStruct((NC, N, D), jnp.float32),
    mesh=_MESH,
    scratch_types=[
        pltpu.VMEM((C,), jnp.int32),
        pltpu.VMEM((C,), jnp.int32),
        pltpu.VMEM((C, D), jnp.float32),
        pltpu.VMEM((ZROWS, D), jnp.float32),
        pltpu.VMEM_SHARED((N, D), jnp.float32),
        pltpu.SemaphoreType.DMA,
    ],
)
def _sc_agg(y_hbm, src_hbm, dst_hbm, out_hbm, si, di, rb, zb, acc, sem):
    cid = lax.axis_index("c")
    sid = lax.axis_index("s")
    w = _worker_id()

    def zfill(i, _):
        def zcol(j, _):
            zb[i, pl.ds(j * 16, 16)] = jnp.zeros((16,), jnp.float32)
            return 0
        return lax.fori_loop(0, D // 16, zcol, 0)

    lax.fori_loop(0, ZROWS, zfill, 0)

    def zstripe(k, _):
        pltpu.sync_copy(zb, acc.at[pl.ds(sid * SPT + k * ZROWS, ZROWS)])
        return 0

    lax.fori_loop(0, SPT // ZROWS, zstripe, 0)

    @pl.when(sid == 0)
    def _():
        pltpu.sync_copy(zb, acc.at[pl.ds(NS * SPT, TAIL)])

    plsc.subcore_barrier()

    def edge_chunk(i, _):
        off = w * EPW + i * C
        pltpu.sync_copy(src_hbm.at[pl.ds(off, C)], si)
        pltpu.sync_copy(dst_hbm.at[pl.ds(off, C)], di)
        pltpu.async_copy(y_hbm.at[si], rb, sem).wait()
        pltpu.sync_copy(rb, acc.at[di], add=True)
        return 0

    lax.fori_loop(0, NCHUNK, edge_chunk, 0)
    plsc.subcore_barrier()

    pltpu.sync_copy(
        acc.at[pl.ds(sid * SPT, SPT)], out_hbm.at[cid, pl.ds(sid * SPT, SPT)]
    )

    @pl.when(sid == 0)
    def _():
        pltpu.sync_copy(
            acc.at[pl.ds(NS * SPT, TAIL)], out_hbm.at[cid, pl.ds(NS * SPT, TAIL)]
        )


@functools.partial(
    pl.kernel,
    out_type=jax.ShapeDtypeStruct((NC, N, D), jnp.float32),
    mesh=_MESH,
    scratch_types=[
        pltpu.VMEM((C,), jnp.int32),
        pltpu.VMEM((C, D), jnp.float32),
        pltpu.VMEM((ZROWS, D), jnp.float32),
        pltpu.VMEM_SHARED((N, D), jnp.float32),
        pltpu.SemaphoreType.DMA,
    ],
)
def _sc_cnt(dst_hbm, out_hbm, di, ob, zb, acc, sem):
    cid = lax.axis_index("c")
    sid = lax.axis_index("s")
    w = _worker_id()

    def fill(i, _):
        def fcol(j, _):
            ob[i, pl.ds(j * 16, 16)] = jnp.ones((16,), jnp.float32)
            return 0
        return lax.fori_loop(0, D // 16, fcol, 0)

    lax.fori_loop(0, C, fill, 0)

    def zfill(i, _):
        def zcol(j, _):
            zb[i, pl.ds(j * 16, 16)] = jnp.zeros((16,), jnp.float32)
            return 0
        return lax.fori_loop(0, D // 16, zcol, 0)

    lax.fori_loop(0, ZROWS, zfill, 0)

    def zstripe(k, _):
        pltpu.sync_copy(zb, acc.at[pl.ds(sid * SPT + k * ZROWS, ZROWS)])
        return 0

    lax.fori_loop(0, SPT // ZROWS, zstripe, 0)

    @pl.when(sid == 0)
    def _():
        pltpu.sync_copy(zb, acc.at[pl.ds(NS * SPT, TAIL)])

    plsc.subcore_barrier()

    def edge_chunk(i, _):
        off = w * EPW + i * C
        pltpu.sync_copy(dst_hbm.at[pl.ds(off, C)], di)
        pltpu.sync_copy(ob, acc.at[di], add=True)
        return 0

    lax.fori_loop(0, NCHUNK, edge_chunk, 0)
    plsc.subcore_barrier()

    pltpu.sync_copy(
        acc.at[pl.ds(sid * SPT, SPT)], out_hbm.at[cid, pl.ds(sid * SPT, SPT)]
    )

    @pl.when(sid == 0)
    def _():
        pltpu.sync_copy(
            acc.at[pl.ds(NS * SPT, TAIL)], out_hbm.at[cid, pl.ds(NS * SPT, TAIL)]
        )


_RB = 1000
_NB = N // _RB


def _tc_pre_body(h_ref, wl_ref, wr_ref, b_ref, y_ref, z_ref):
    h = h_ref[...]
    dn = (((1,), (1,)), ((), ()))
    y_ref[...] = lax.dot_general(h, wl_ref[...], dn, preferred_element_type=jnp.float32)
    z_ref[...] = (
        lax.dot_general(h, wr_ref[...], dn, preferred_element_type=jnp.float32)
        + b_ref[...]
    )


_tc_pre = pl.pallas_call(
    _tc_pre_body,
    grid=(_NB,),
    in_specs=[
        pl.BlockSpec((_RB, D), lambda i: (i, 0)),
        pl.BlockSpec((D, D), lambda i: (0, 0)),
        pl.BlockSpec((D, D), lambda i: (0, 0)),
        pl.BlockSpec((1, D), lambda i: (0, 0)),
    ],
    out_specs=[
        pl.BlockSpec((_RB, D), lambda i: (i, 0)),
        pl.BlockSpec((_RB, D), lambda i: (i, 0)),
    ],
    out_shape=[
        jax.ShapeDtypeStruct((N, D), jnp.float32),
        jax.ShapeDtypeStruct((N, D), jnp.float32),
    ],
)


def _tc_post_body(relu, p_ref, c_ref, z_ref, o_ref):
    p = p_ref[0] + p_ref[1]
    cnt = (c_ref[0] + c_ref[1])[:, 0:1]
    inv = 1.0 / jnp.maximum(cnt, 1.0)
    out = p * inv + z_ref[...]
    if relu:
        out = jnp.maximum(out, 0.0)
    o_ref[...] = out


def _make_tc_post(relu):
    return pl.pallas_call(
        functools.partial(_tc_post_body, relu),
        grid=(_NB,),
        in_specs=[
            pl.BlockSpec((NC, _RB, D), lambda i: (0, i, 0)),
            pl.BlockSpec((NC, _RB, D), lambda i: (0, i, 0)),
            pl.BlockSpec((_RB, D), lambda i: (i, 0)),
        ],
        out_specs=pl.BlockSpec((_RB, D), lambda i: (i, 0)),
        out_shape=jax.ShapeDtypeStruct((N, D), jnp.float32),
    )


_tc_post_relu = _make_tc_post(True)
_tc_post_last = _make_tc_post(False)


def kernel(x, edge_index, Wl0, Wr0, b0, Wl1, Wr1, b1, Wl2, Wr2, b2):
    src = edge_index[0].astype(jnp.int32)
    dst = edge_index[1].astype(jnp.int32)
    cnt_parts = _sc_cnt(dst)
    h = x
    for i, (Wl, Wr, b) in enumerate([(Wl0, Wr0, b0), (Wl1, Wr1, b1), (Wl2, Wr2, b2)]):
        y, z = _tc_pre(h, Wl, Wr, b.reshape(1, D))
        parts = _sc_agg(y, src, dst)
        post = _tc_post_relu if i < 2 else _tc_post_last
        h = post(parts, cnt_parts, z)
    return h


# speedup vs baseline: 8.4490x; 1.8011x over previous
import functools

import jax
import jax.numpy as jnp
from jax import lax
from jax.experimental import pallas as pl
from jax.experimental.pallas import tpu as pltpu
from jax.experimental.pallas import tpu_sc as plsc

N = 10000
E = 320000
D = 128

NC = 2
NS = 16
NW = NC * NS
EPW = E // NW
C = 80
NCHUNK = EPW // C
PAIRS = (NCHUNK - 1) // 2
SPT = 624
TAIL = N - NS * SPT
ZROWS = 16

_MESH = plsc.VectorSubcoreMesh(
    core_axis_name="c", subcore_axis_name="s", num_cores=NC, num_subcores=NS
)


def _worker_id():
    return lax.axis_index("c") * NS + lax.axis_index("s")


@functools.partial(
    pl.kernel,
    out_type=jax.ShapeDtypeStruct((NC, N, D), jnp.float32),
    mesh=_MESH,
    scratch_types=[
        pltpu.VMEM((EPW,), jnp.int32),
        pltpu.VMEM((EPW,), jnp.int32),
        pltpu.VMEM((C, D), jnp.float32),
        pltpu.VMEM((C, D), jnp.float32),
        pltpu.VMEM((ZROWS, D), jnp.float32),
        pltpu.VMEM_SHARED((N, D), jnp.float32),
        pltpu.SemaphoreType.DMA,
        pltpu.SemaphoreType.DMA,
        pltpu.SemaphoreType.DMA,
        pltpu.SemaphoreType.DMA,
    ],
)
def _sc_agg(y_hbm, src_hbm, dst_hbm, out_hbm, sia, dia, rb0, rb1, zb, acc,
            sg0, sg1, ss0, ss1):
    cid = lax.axis_index("c")
    sid = lax.axis_index("s")
    w = _worker_id()

    def zfill(i, _):
        def zcol(j, _):
            zb[i, pl.ds(j * 16, 16)] = jnp.zeros((16,), jnp.float32)
            return 0
        return lax.fori_loop(0, D // 16, zcol, 0)

    lax.fori_loop(0, ZROWS, zfill, 0)

    def zstripe(k, _):
        pltpu.sync_copy(zb, acc.at[pl.ds(sid * SPT + k * ZROWS, ZROWS)])
        return 0

    lax.fori_loop(0, SPT // ZROWS, zstripe, 0)

    @pl.when(sid == 0)
    def _():
        pltpu.sync_copy(zb, acc.at[pl.ds(NS * SPT, TAIL)])

    pltpu.sync_copy(src_hbm.at[pl.ds(w * EPW, EPW)], sia)
    pltpu.sync_copy(dst_hbm.at[pl.ds(w * EPW, EPW)], dia)
    plsc.subcore_barrier()

    def gat(c, rb, sem):
        return pltpu.make_async_copy(y_hbm.at[sia.at[pl.ds(c * C, C)]], rb, sem)

    def sca(c, rb, sem):
        return pltpu.make_async_copy(rb, acc.at[dia.at[pl.ds(c * C, C)]], sem)

    gat(0, rb0, sg0).start()

    def pair(g, _):
        c0 = 2 * g
        gat(c0, rb0, sg0).wait()
        sca(c0, rb0, ss0).start(add=True)

        @pl.when(g > 0)
        def _():
            sca(c0 - 1, rb1, ss1).wait()

        gat(c0 + 1, rb1, sg1).start()
        gat(c0 + 1, rb1, sg1).wait()
        sca(c0 + 1, rb1, ss1).start(add=True)
        sca(c0, rb0, ss0).wait()
        gat(c0 + 2, rb0, sg0).start()
        return 0

    lax.fori_loop(0, PAIRS, pair, 0)

    last = 2 * PAIRS
    gat(last, rb0, sg0).wait()
    sca(last, rb0, ss0).start(add=True)
    sca(last - 1, rb1, ss1).wait()
    sca(last, rb0, ss0).wait()
    plsc.subcore_barrier()

    pltpu.sync_copy(
        acc.at[pl.ds(sid * SPT, SPT)], out_hbm.at[cid, pl.ds(sid * SPT, SPT)]
    )

    @pl.when(sid == 0)
    def _():
        pltpu.sync_copy(
            acc.at[pl.ds(NS * SPT, TAIL)], out_hbm.at[cid, pl.ds(NS * SPT, TAIL)]
        )


@functools.partial(
    pl.kernel,
    out_type=jax.ShapeDtypeStruct((NC, N, D), jnp.float32),
    mesh=_MESH,
    scratch_types=[
        pltpu.VMEM((EPW,), jnp.int32),
        pltpu.VMEM((C, D), jnp.float32),
        pltpu.VMEM((ZROWS, D), jnp.float32),
        pltpu.VMEM_SHARED((N, D), jnp.float32),
        pltpu.SemaphoreType.DMA,
    ],
)
def _sc_cnt(dst_hbm, out_hbm, dia, ob, zb, acc, sem):
    cid = lax.axis_index("c")
    sid = lax.axis_index("s")
    w = _worker_id()

    def fill(i, _):
        def fcol(j, _):
            ob[i, pl.ds(j * 16, 16)] = jnp.ones((16,), jnp.float32)
            return 0
        return lax.fori_loop(0, D // 16, fcol, 0)

    lax.fori_loop(0, C, fill, 0)

    def zfill(i, _):
        def zcol(j, _):
            zb[i, pl.ds(j * 16, 16)] = jnp.zeros((16,), jnp.float32)
            return 0
        return lax.fori_loop(0, D // 16, zcol, 0)

    lax.fori_loop(0, ZROWS, zfill, 0)

    def zstripe(k, _):
        pltpu.sync_copy(zb, acc.at[pl.ds(sid * SPT + k * ZROWS, ZROWS)])
        return 0

    lax.fori_loop(0, SPT // ZROWS, zstripe, 0)

    @pl.when(sid == 0)
    def _():
        pltpu.sync_copy(zb, acc.at[pl.ds(NS * SPT, TAIL)])

    pltpu.sync_copy(dst_hbm.at[pl.ds(w * EPW, EPW)], dia)
    plsc.subcore_barrier()

    def sca(c):
        return pltpu.make_async_copy(ob, acc.at[dia.at[pl.ds(c * C, C)]], sem)

    def group(gq, _):
        for j in range(5):
            sca(gq * 5 + j).start(add=True)
        for j in range(5):
            sca(gq * 5 + j).wait()
        return 0

    lax.fori_loop(0, NCHUNK // 5, group, 0)
    plsc.subcore_barrier()

    pltpu.sync_copy(
        acc.at[pl.ds(sid * SPT, SPT)], out_hbm.at[cid, pl.ds(sid * SPT, SPT)]
    )

    @pl.when(sid == 0)
    def _():
        pltpu.sync_copy(
            acc.at[pl.ds(NS * SPT, TAIL)], out_hbm.at[cid, pl.ds(NS * SPT, TAIL)]
        )


_RB = 1000
_NB = N // _RB


def _tc_pre_body(h_ref, wl_ref, wr_ref, b_ref, y_ref, z_ref):
    h = h_ref[...]
    dn = (((1,), (1,)), ((), ()))
    y_ref[...] = lax.dot_general(h, wl_ref[...], dn, preferred_element_type=jnp.float32)
    z_ref[...] = (
        lax.dot_general(h, wr_ref[...], dn, preferred_element_type=jnp.float32)
        + b_ref[...]
    )


_tc_pre = pl.pallas_call(
    _tc_pre_body,
    grid=(_NB,),
    in_specs=[
        pl.BlockSpec((_RB, D), lambda i: (i, 0)),
        pl.BlockSpec((D, D), lambda i: (0, 0)),
        pl.BlockSpec((D, D), lambda i: (0, 0)),
        pl.BlockSpec((1, D), lambda i: (0, 0)),
    ],
    out_specs=[
        pl.BlockSpec((_RB, D), lambda i: (i, 0)),
        pl.BlockSpec((_RB, D), lambda i: (i, 0)),
    ],
    out_shape=[
        jax.ShapeDtypeStruct((N, D), jnp.float32),
        jax.ShapeDtypeStruct((N, D), jnp.float32),
    ],
)


def _tc_post_body(relu, p_ref, c_ref, z_ref, o_ref):
    p = p_ref[0] + p_ref[1]
    cnt = (c_ref[0] + c_ref[1])[:, 0:1]
    inv = 1.0 / jnp.maximum(cnt, 1.0)
    out = p * inv + z_ref[...]
    if relu:
        out = jnp.maximum(out, 0.0)
    o_ref[...] = out


def _make_tc_post(relu):
    return pl.pallas_call(
        functools.partial(_tc_post_body, relu),
        grid=(_NB,),
        in_specs=[
            pl.BlockSpec((NC, _RB, D), lambda i: (0, i, 0)),
            pl.BlockSpec((NC, _RB, D), lambda i: (0, i, 0)),
            pl.BlockSpec((_RB, D), lambda i: (i, 0)),
        ],
        out_specs=pl.BlockSpec((_RB, D), lambda i: (i, 0)),
        out_shape=jax.ShapeDtypeStruct((N, D), jnp.float32),
    )


_tc_post_relu = _make_tc_post(True)
_tc_post_last = _make_tc_post(False)


def kernel(x, edge_index, Wl0, Wr0, b0, Wl1, Wr1, b1, Wl2, Wr2, b2):
    src = edge_index[0].astype(jnp.int32)
    dst = edge_index[1].astype(jnp.int32)
    cnt_parts = _sc_cnt(dst)
    h = x
    for i, (Wl, Wr, b) in enumerate([(Wl0, Wr0, b0), (Wl1, Wr1, b1), (Wl2, Wr2, b2)]):
        y, z = _tc_pre(h, Wl, Wr, b.reshape(1, D))
        parts = _sc_agg(y, src, dst)
        post = _tc_post_relu if i < 2 else _tc_post_last
        h = post(parts, cnt_parts, z)
    return h


# speedup vs baseline: 10.8039x; 1.2787x over previous
import functools

import jax
import jax.numpy as jnp
from jax import lax
from jax.experimental import pallas as pl
from jax.experimental.pallas import tpu as pltpu
from jax.experimental.pallas import tpu_sc as plsc

N = 10000
E = 320000
D = 128

NC = 2
NS = 16
NW = NC * NS
EPW = E // NW
C = 40
NCHUNK = EPW // C
NB = 4
CC = 80
NCHUNKC = EPW // CC
SPT = 624
TAIL = N - NS * SPT
ZROWS = 8

_MESH = plsc.VectorSubcoreMesh(
    core_axis_name="c", subcore_axis_name="s", num_cores=NC, num_subcores=NS
)


def _worker_id():
    return lax.axis_index("c") * NS + lax.axis_index("s")


@functools.partial(
    pl.kernel,
    out_type=jax.ShapeDtypeStruct((NC, N, D), jnp.float32),
    mesh=_MESH,
    scratch_types=[
        pltpu.VMEM((EPW,), jnp.int32),
        pltpu.VMEM((EPW,), jnp.int32),
        [pltpu.VMEM((C, D), jnp.float32)] * NB,
        pltpu.VMEM((ZROWS, D), jnp.float32),
        pltpu.VMEM_SHARED((N, D), jnp.float32),
        [pltpu.SemaphoreType.DMA] * NB,
        [pltpu.SemaphoreType.DMA] * NB,
    ],
)
def _sc_agg(y_hbm, src_hbm, dst_hbm, out_hbm, sia, dia, rbs, zb, acc, sgs, sss):
    cid = lax.axis_index("c")
    sid = lax.axis_index("s")
    w = _worker_id()

    def zfill(i, _):
        def zcol(j, _):
            zb[i, pl.ds(j * 16, 16)] = jnp.zeros((16,), jnp.float32)
            return 0
        return lax.fori_loop(0, D // 16, zcol, 0)

    lax.fori_loop(0, ZROWS, zfill, 0)

    def zstripe(k, _):
        pltpu.sync_copy(zb, acc.at[pl.ds(sid * SPT + k * ZROWS, ZROWS)])
        return 0

    lax.fori_loop(0, SPT // ZROWS, zstripe, 0)

    @pl.when(sid == 0)
    def _():
        def ztail(k, _):
            pltpu.sync_copy(zb, acc.at[pl.ds(NS * SPT + k * ZROWS, ZROWS)])
            return 0
        lax.fori_loop(0, TAIL // ZROWS, ztail, 0)

    pltpu.sync_copy(src_hbm.at[pl.ds(w * EPW, EPW)], sia)
    pltpu.sync_copy(dst_hbm.at[pl.ds(w * EPW, EPW)], dia)
    plsc.subcore_barrier()

    def gat(c, b):
        return pltpu.make_async_copy(
            y_hbm.at[sia.at[pl.ds(c * C, C)]], rbs[b], sgs[b]
        )

    def sca(c, b):
        return pltpu.make_async_copy(
            rbs[b], acc.at[dia.at[pl.ds(c * C, C)]], sss[b]
        )

    gat(0, 0).start()
    gat(1, 1).start()

    def group(g, _):
        for j in range(NB):
            c = NB * g + 2 + j
            b = (2 + j) % NB

            @pl.when(c >= NB)
            def _():
                sca(c - NB, b).wait()

            gat(c, b).start()
            gat(c - 2, j).wait()
            sca(c - 2, j).start(add=True)
        return 0

    lax.fori_loop(0, (NCHUNK - 2) // NB, group, 0)

    for c in (NCHUNK - 2, NCHUNK - 1):
        gat(c, c % NB).wait()
        sca(c, c % NB).start(add=True)
    for c in range(NCHUNK - NB, NCHUNK):
        sca(c, c % NB).wait()
    plsc.subcore_barrier()

    pltpu.sync_copy(
        acc.at[pl.ds(sid * SPT, SPT)], out_hbm.at[cid, pl.ds(sid * SPT, SPT)]
    )

    @pl.when(sid == 0)
    def _():
        pltpu.sync_copy(
            acc.at[pl.ds(NS * SPT, TAIL)], out_hbm.at[cid, pl.ds(NS * SPT, TAIL)]
        )


@functools.partial(
    pl.kernel,
    out_type=jax.ShapeDtypeStruct((NC, N, D), jnp.float32),
    mesh=_MESH,
    scratch_types=[
        pltpu.VMEM((EPW,), jnp.int32),
        pltpu.VMEM((CC, D), jnp.float32),
        pltpu.VMEM((ZROWS, D), jnp.float32),
        pltpu.VMEM_SHARED((N, D), jnp.float32),
        pltpu.SemaphoreType.DMA,
    ],
)
def _sc_cnt(dst_hbm, out_hbm, dia, ob, zb, acc, sem):
    cid = lax.axis_index("c")
    sid = lax.axis_index("s")
    w = _worker_id()

    def fill(i, _):
        def fcol(j, _):
            ob[i, pl.ds(j * 16, 16)] = jnp.ones((16,), jnp.float32)
            return 0
        return lax.fori_loop(0, D // 16, fcol, 0)

    lax.fori_loop(0, CC, fill, 0)

    def zfill(i, _):
        def zcol(j, _):
            zb[i, pl.ds(j * 16, 16)] = jnp.zeros((16,), jnp.float32)
            return 0
        return lax.fori_loop(0, D // 16, zcol, 0)

    lax.fori_loop(0, ZROWS, zfill, 0)

    def zstripe(k, _):
        pltpu.sync_copy(zb, acc.at[pl.ds(sid * SPT + k * ZROWS, ZROWS)])
        return 0

    lax.fori_loop(0, SPT // ZROWS, zstripe, 0)

    @pl.when(sid == 0)
    def _():
        def ztail(k, _):
            pltpu.sync_copy(zb, acc.at[pl.ds(NS * SPT + k * ZROWS, ZROWS)])
            return 0
        lax.fori_loop(0, TAIL // ZROWS, ztail, 0)

    pltpu.sync_copy(dst_hbm.at[pl.ds(w * EPW, EPW)], dia)
    plsc.subcore_barrier()

    def sca(c):
        return pltpu.make_async_copy(ob, acc.at[dia.at[pl.ds(c * CC, CC)]], sem)

    def group(gq, _):
        for j in range(5):
            sca(gq * 5 + j).start(add=True)
        for j in range(5):
            sca(gq * 5 + j).wait()
        return 0

    lax.fori_loop(0, NCHUNKC // 5, group, 0)
    plsc.subcore_barrier()

    pltpu.sync_copy(
        acc.at[pl.ds(sid * SPT, SPT)], out_hbm.at[cid, pl.ds(sid * SPT, SPT)]
    )

    @pl.when(sid == 0)
    def _():
        pltpu.sync_copy(
            acc.at[pl.ds(NS * SPT, TAIL)], out_hbm.at[cid, pl.ds(NS * SPT, TAIL)]
        )


_RB = 1000
_NB = N // _RB


def _tc_pre_body(h_ref, wl_ref, wr_ref, b_ref, y_ref, z_ref):
    h = h_ref[...]
    dn = (((1,), (1,)), ((), ()))
    y_ref[...] = lax.dot_general(h, wl_ref[...], dn, preferred_element_type=jnp.float32)
    z_ref[...] = (
        lax.dot_general(h, wr_ref[...], dn, preferred_element_type=jnp.float32)
        + b_ref[...]
    )


_tc_pre = pl.pallas_call(
    _tc_pre_body,
    grid=(_NB,),
    in_specs=[
        pl.BlockSpec((_RB, D), lambda i: (i, 0)),
        pl.BlockSpec((D, D), lambda i: (0, 0)),
        pl.BlockSpec((D, D), lambda i: (0, 0)),
        pl.BlockSpec((1, D), lambda i: (0, 0)),
    ],
    out_specs=[
        pl.BlockSpec((_RB, D), lambda i: (i, 0)),
        pl.BlockSpec((_RB, D), lambda i: (i, 0)),
    ],
    out_shape=[
        jax.ShapeDtypeStruct((N, D), jnp.float32),
        jax.ShapeDtypeStruct((N, D), jnp.float32),
    ],
)


def _tc_post_body(relu, p_ref, c_ref, z_ref, o_ref):
    p = p_ref[0] + p_ref[1]
    cnt = (c_ref[0] + c_ref[1])[:, 0:1]
    inv = 1.0 / jnp.maximum(cnt, 1.0)
    out = p * inv + z_ref[...]
    if relu:
        out = jnp.maximum(out, 0.0)
    o_ref[...] = out


def _make_tc_post(relu):
    return pl.pallas_call(
        functools.partial(_tc_post_body, relu),
        grid=(_NB,),
        in_specs=[
            pl.BlockSpec((NC, _RB, D), lambda i: (0, i, 0)),
            pl.BlockSpec((NC, _RB, D), lambda i: (0, i, 0)),
            pl.BlockSpec((_RB, D), lambda i: (i, 0)),
        ],
        out_specs=pl.BlockSpec((_RB, D), lambda i: (i, 0)),
        out_shape=jax.ShapeDtypeStruct((N, D), jnp.float32),
    )


_tc_post_relu = _make_tc_post(True)
_tc_post_last = _make_tc_post(False)


def kernel(x, edge_index, Wl0, Wr0, b0, Wl1, Wr1, b1, Wl2, Wr2, b2):
    src = edge_index[0].astype(jnp.int32)
    dst = edge_index[1].astype(jnp.int32)
    cnt_parts = _sc_cnt(dst)
    h = x
    for i, (Wl, Wr, b) in enumerate([(Wl0, Wr0, b0), (Wl1, Wr1, b1), (Wl2, Wr2, b2)]):
        y, z = _tc_pre(h, Wl, Wr, b.reshape(1, D))
        parts = _sc_agg(y, src, dst)
        post = _tc_post_relu if i < 2 else _tc_post_last
        h = post(parts, cnt_parts, z)
    return h


# speedup vs baseline: 11.1805x; 1.0349x over previous
import functools

import jax
import jax.numpy as jnp
from jax import lax
from jax.experimental import pallas as pl
from jax.experimental.pallas import tpu as pltpu
from jax.experimental.pallas import tpu_sc as plsc

N = 10000
E = 320000
D = 128

NC = 2
NS = 16
NW = NC * NS
EPW = E // NW
C = 40
NCHUNK = EPW // C
NB = 4
CC = 80
NCHUNKC = EPW // CC
SPT = 624
TAIL = N - NS * SPT
ZROWS = 8

_MESH = plsc.VectorSubcoreMesh(
    core_axis_name="c", subcore_axis_name="s", num_cores=NC, num_subcores=NS
)


def _worker_id():
    return lax.axis_index("c") * NS + lax.axis_index("s")


@functools.partial(
    pl.kernel,
    out_type=jax.ShapeDtypeStruct((NC, N, D), jnp.float32),
    mesh=_MESH,
    scratch_types=[
        pltpu.VMEM((EPW,), jnp.int32),
        pltpu.VMEM((EPW,), jnp.int32),
        [pltpu.VMEM((C, D), jnp.float32)] * NB,
        pltpu.VMEM((ZROWS, D), jnp.float32),
        pltpu.VMEM_SHARED((N, D), jnp.float32),
        [pltpu.SemaphoreType.DMA] * NB,
        [pltpu.SemaphoreType.DMA] * NB,
    ],
)
def _sc_agg(y_hbm, src_hbm, dst_hbm, out_hbm, sia, dia, rbs, zb, acc, sgs, sss):
    cid = lax.axis_index("c")
    sid = lax.axis_index("s")
    w = _worker_id()

    def zfill(i, _):
        def zcol(j, _):
            zb[i, pl.ds(j * 16, 16)] = jnp.zeros((16,), jnp.float32)
            return 0
        return lax.fori_loop(0, D // 16, zcol, 0)

    lax.fori_loop(0, ZROWS, zfill, 0)

    def zstripe(k, _):
        pltpu.sync_copy(zb, acc.at[pl.ds(sid * SPT + k * ZROWS, ZROWS)])
        return 0

    lax.fori_loop(0, SPT // ZROWS, zstripe, 0)

    @pl.when(sid == 0)
    def _():
        def ztail(k, _):
            pltpu.sync_copy(zb, acc.at[pl.ds(NS * SPT + k * ZROWS, ZROWS)])
            return 0
        lax.fori_loop(0, TAIL // ZROWS, ztail, 0)

    pltpu.sync_copy(src_hbm.at[pl.ds(w * EPW, EPW)], sia)
    pltpu.sync_copy(dst_hbm.at[pl.ds(w * EPW, EPW)], dia)
    plsc.subcore_barrier()

    def gat(c, b):
        return pltpu.make_async_copy(
            y_hbm.at[sia.at[pl.ds(c * C, C)]], rbs[b], sgs[b]
        )

    def sca(c, b):
        return pltpu.make_async_copy(
            rbs[b], acc.at[dia.at[pl.ds(c * C, C)]], sss[b]
        )

    gat(0, 0).start()
    gat(1, 1).start()

    def group(g, _):
        for j in range(NB):
            c = NB * g + 2 + j
            b = (2 + j) % NB

            @pl.when(c >= NB)
            def _():
                sca(c - NB, b).wait()

            gat(c, b).start()
            gat(c - 2, j).wait()
            sca(c - 2, j).start(add=True)
        return 0

    lax.fori_loop(0, (NCHUNK - 2) // NB, group, 0)

    for c in (NCHUNK - 2, NCHUNK - 1):
        gat(c, c % NB).wait()
        sca(c, c % NB).start(add=True)
    for c in range(NCHUNK - NB, NCHUNK):
        sca(c, c % NB).wait()
    plsc.subcore_barrier()

    pltpu.sync_copy(
        acc.at[pl.ds(sid * SPT, SPT)], out_hbm.at[cid, pl.ds(sid * SPT, SPT)]
    )

    @pl.when(sid == 0)
    def _():
        pltpu.sync_copy(
            acc.at[pl.ds(NS * SPT, TAIL)], out_hbm.at[cid, pl.ds(NS * SPT, TAIL)]
        )


@functools.partial(
    pl.kernel,
    out_type=jax.ShapeDtypeStruct((NC, N, D), jnp.float32),
    mesh=_MESH,
    scratch_types=[
        pltpu.VMEM((EPW,), jnp.int32),
        pltpu.VMEM((CC, D), jnp.float32),
        pltpu.VMEM((ZROWS, D), jnp.float32),
        pltpu.VMEM_SHARED((N, D), jnp.float32),
        pltpu.SemaphoreType.DMA,
    ],
)
def _sc_cnt(dst_hbm, out_hbm, dia, ob, zb, acc, sem):
    cid = lax.axis_index("c")
    sid = lax.axis_index("s")
    w = _worker_id()

    def fill(i, _):
        def fcol(j, _):
            ob[i, pl.ds(j * 16, 16)] = jnp.ones((16,), jnp.float32)
            return 0
        return lax.fori_loop(0, D // 16, fcol, 0)

    lax.fori_loop(0, CC, fill, 0)

    def zfill(i, _):
        def zcol(j, _):
            zb[i, pl.ds(j * 16, 16)] = jnp.zeros((16,), jnp.float32)
            return 0
        return lax.fori_loop(0, D // 16, zcol, 0)

    lax.fori_loop(0, ZROWS, zfill, 0)

    def zstripe(k, _):
        pltpu.sync_copy(zb, acc.at[pl.ds(sid * SPT + k * ZROWS, ZROWS)])
        return 0

    lax.fori_loop(0, SPT // ZROWS, zstripe, 0)

    @pl.when(sid == 0)
    def _():
        def ztail(k, _):
            pltpu.sync_copy(zb, acc.at[pl.ds(NS * SPT + k * ZROWS, ZROWS)])
            return 0
        lax.fori_loop(0, TAIL // ZROWS, ztail, 0)

    pltpu.sync_copy(dst_hbm.at[pl.ds(w * EPW, EPW)], dia)
    plsc.subcore_barrier()

    def sca(c):
        return pltpu.make_async_copy(ob, acc.at[dia.at[pl.ds(c * CC, CC)]], sem)

    def group(gq, _):
        for j in range(5):
            sca(gq * 5 + j).start(add=True)
        for j in range(5):
            sca(gq * 5 + j).wait()
        return 0

    lax.fori_loop(0, NCHUNKC // 5, group, 0)
    plsc.subcore_barrier()

    pltpu.sync_copy(
        acc.at[pl.ds(sid * SPT, SPT)], out_hbm.at[cid, pl.ds(sid * SPT, SPT)]
    )

    @pl.when(sid == 0)
    def _():
        pltpu.sync_copy(
            acc.at[pl.ds(NS * SPT, TAIL)], out_hbm.at[cid, pl.ds(NS * SPT, TAIL)]
        )


_RB = 1000
_NB = N // _RB


def _tc_pre_body(h_ref, wl_ref, wr_ref, b_ref, y_ref, z_ref):
    h = h_ref[...]
    dn = (((1,), (1,)), ((), ()))
    y_ref[...] = lax.dot_general(h, wl_ref[...], dn, preferred_element_type=jnp.float32)
    z_ref[...] = (
        lax.dot_general(h, wr_ref[...], dn, preferred_element_type=jnp.float32)
        + b_ref[...]
    )


_tc_pre = pl.pallas_call(
    _tc_pre_body,
    grid=(_NB,),
    in_specs=[
        pl.BlockSpec((_RB, D), lambda i: (i, 0)),
        pl.BlockSpec((D, D), lambda i: (0, 0)),
        pl.BlockSpec((D, D), lambda i: (0, 0)),
        pl.BlockSpec((1, D), lambda i: (0, 0)),
    ],
    out_specs=[
        pl.BlockSpec((_RB, D), lambda i: (i, 0)),
        pl.BlockSpec((_RB, D), lambda i: (i, 0)),
    ],
    out_shape=[
        jax.ShapeDtypeStruct((N, D), jnp.float32),
        jax.ShapeDtypeStruct((N, D), jnp.float32),
    ],
)


def _tc_combo_body(p_ref, c_ref, z_ref, wl_ref, wr_ref, b_ref, y_ref, z2_ref):
    p = p_ref[0] + p_ref[1]
    cnt = (c_ref[0] + c_ref[1])[:, 0:1]
    inv = 1.0 / jnp.maximum(cnt, 1.0)
    h = jnp.maximum(p * inv + z_ref[...], 0.0)
    dn = (((1,), (1,)), ((), ()))
    y_ref[...] = lax.dot_general(h, wl_ref[...], dn, preferred_element_type=jnp.float32)
    z2_ref[...] = (
        lax.dot_general(h, wr_ref[...], dn, preferred_element_type=jnp.float32)
        + b_ref[...]
    )


_tc_combo = pl.pallas_call(
    _tc_combo_body,
    grid=(_NB,),
    in_specs=[
        pl.BlockSpec((NC, _RB, D), lambda i: (0, i, 0)),
        pl.BlockSpec((NC, _RB, D), lambda i: (0, i, 0)),
        pl.BlockSpec((_RB, D), lambda i: (i, 0)),
        pl.BlockSpec((D, D), lambda i: (0, 0)),
        pl.BlockSpec((D, D), lambda i: (0, 0)),
        pl.BlockSpec((1, D), lambda i: (0, 0)),
    ],
    out_specs=[
        pl.BlockSpec((_RB, D), lambda i: (i, 0)),
        pl.BlockSpec((_RB, D), lambda i: (i, 0)),
    ],
    out_shape=[
        jax.ShapeDtypeStruct((N, D), jnp.float32),
        jax.ShapeDtypeStruct((N, D), jnp.float32),
    ],
)


def _tc_post_body(relu, p_ref, c_ref, z_ref, o_ref):
    p = p_ref[0] + p_ref[1]
    cnt = (c_ref[0] + c_ref[1])[:, 0:1]
    inv = 1.0 / jnp.maximum(cnt, 1.0)
    out = p * inv + z_ref[...]
    if relu:
        out = jnp.maximum(out, 0.0)
    o_ref[...] = out


def _make_tc_post(relu):
    return pl.pallas_call(
        functools.partial(_tc_post_body, relu),
        grid=(_NB,),
        in_specs=[
            pl.BlockSpec((NC, _RB, D), lambda i: (0, i, 0)),
            pl.BlockSpec((NC, _RB, D), lambda i: (0, i, 0)),
            pl.BlockSpec((_RB, D), lambda i: (i, 0)),
        ],
        out_specs=pl.BlockSpec((_RB, D), lambda i: (i, 0)),
        out_shape=jax.ShapeDtypeStruct((N, D), jnp.float32),
    )


_tc_post_relu = _make_tc_post(True)
_tc_post_last = _make_tc_post(False)


def kernel(x, edge_index, Wl0, Wr0, b0, Wl1, Wr1, b1, Wl2, Wr2, b2):
    src = edge_index[0].astype(jnp.int32)
    dst = edge_index[1].astype(jnp.int32)
    cnt_parts = _sc_cnt(dst)
    y, z = _tc_pre(x, Wl0, Wr0, b0.reshape(1, D))
    for Wl, Wr, b in [(Wl1, Wr1, b1), (Wl2, Wr2, b2)]:
        parts = _sc_agg(y, src, dst)
        y, z = _tc_combo(parts, cnt_parts, z, Wl, Wr, b.reshape(1, D))
    parts = _sc_agg(y, src, dst)
    return _tc_post_last(parts, cnt_parts, z)


# speedup vs baseline: 11.8285x; 1.0580x over previous
import functools

import jax
import jax.numpy as jnp
from jax import lax
from jax.experimental import pallas as pl
from jax.experimental.pallas import tpu as pltpu
from jax.experimental.pallas import tpu_sc as plsc

N = 10000
E = 320000
D = 128

NC = 2
NS = 16
NW = NC * NS
EPW = E // NW
C = 40
NCHUNK = EPW // C
NB = 5
GL = 3
CC = 80
NCHUNKC = EPW // CC
SPT = 624
TAIL = N - NS * SPT
ZROWS = 8

_MESH = plsc.VectorSubcoreMesh(
    core_axis_name="c", subcore_axis_name="s", num_cores=NC, num_subcores=NS
)


def _worker_id():
    return lax.axis_index("c") * NS + lax.axis_index("s")


@functools.partial(
    pl.kernel,
    out_type=jax.ShapeDtypeStruct((NC, N, D), jnp.float32),
    mesh=_MESH,
    scratch_types=[
        pltpu.VMEM((EPW,), jnp.int32),
        pltpu.VMEM((EPW,), jnp.int32),
        [pltpu.VMEM((C, D), jnp.float32)] * NB,
        pltpu.VMEM((ZROWS, D), jnp.float32),
        pltpu.VMEM_SHARED((N, D), jnp.float32),
        [pltpu.SemaphoreType.DMA] * NB,
        [pltpu.SemaphoreType.DMA] * NB,
    ],
)
def _sc_agg(y_hbm, src_hbm, dst_hbm, out_hbm, sia, dia, rbs, zb, acc, sgs, sss):
    cid = lax.axis_index("c")
    sid = lax.axis_index("s")
    w = _worker_id()

    def zfill(i, _):
        def zcol(j, _):
            zb[i, pl.ds(j * 16, 16)] = jnp.zeros((16,), jnp.float32)
            return 0
        return lax.fori_loop(0, D // 16, zcol, 0)

    lax.fori_loop(0, ZROWS, zfill, 0)

    def zstripe(k, _):
        pltpu.sync_copy(zb, acc.at[pl.ds(sid * SPT + k * ZROWS, ZROWS)])
        return 0

    lax.fori_loop(0, SPT // ZROWS, zstripe, 0)

    @pl.when(sid == 0)
    def _():
        def ztail(k, _):
            pltpu.sync_copy(zb, acc.at[pl.ds(NS * SPT + k * ZROWS, ZROWS)])
            return 0
        lax.fori_loop(0, TAIL // ZROWS, ztail, 0)

    pltpu.sync_copy(src_hbm.at[pl.ds(w * EPW, EPW)], sia)
    pltpu.sync_copy(dst_hbm.at[pl.ds(w * EPW, EPW)], dia)
    plsc.subcore_barrier()

    def gat(c, b):
        return pltpu.make_async_copy(
            y_hbm.at[sia.at[pl.ds(c * C, C)]], rbs[b], sgs[b]
        )

    def sca(c, b):
        return pltpu.make_async_copy(
            rbs[b], acc.at[dia.at[pl.ds(c * C, C)]], sss[b]
        )

    for c in range(GL):
        gat(c, c).start()

    STEPS = NCHUNK - GL
    MAIN = STEPS // NB
    REM = STEPS - MAIN * NB

    def group(g, _):
        for j in range(NB):
            c = NB * g + GL + j
            b = (GL + j) % NB

            @pl.when(c >= NB)
            def _():
                sca(c - NB, b).wait()

            gat(c, b).start()
            gat(c - GL, j).wait()
            sca(c - GL, j).start(add=True)
        return 0

    lax.fori_loop(0, MAIN, group, 0)

    for k in range(REM):
        c = MAIN * NB + GL + k
        sca(c - NB, c % NB).wait()
        gat(c, c % NB).start()
        gat(c - GL, (c - GL) % NB).wait()
        sca(c - GL, (c - GL) % NB).start(add=True)
    for c in range(NCHUNK - GL, NCHUNK):
        gat(c, c % NB).wait()
        sca(c, c % NB).start(add=True)
    for c in range(NCHUNK - NB, NCHUNK):
        sca(c, c % NB).wait()
    plsc.subcore_barrier()

    pltpu.sync_copy(
        acc.at[pl.ds(sid * SPT, SPT)], out_hbm.at[cid, pl.ds(sid * SPT, SPT)]
    )

    @pl.when(sid == 0)
    def _():
        pltpu.sync_copy(
            acc.at[pl.ds(NS * SPT, TAIL)], out_hbm.at[cid, pl.ds(NS * SPT, TAIL)]
        )


@functools.partial(
    pl.kernel,
    out_type=jax.ShapeDtypeStruct((NC, N, D), jnp.float32),
    mesh=_MESH,
    scratch_types=[
        pltpu.VMEM((EPW,), jnp.int32),
        pltpu.VMEM((CC, D), jnp.float32),
        pltpu.VMEM((ZROWS, D), jnp.float32),
        pltpu.VMEM_SHARED((N, D), jnp.float32),
        pltpu.SemaphoreType.DMA,
    ],
)
def _sc_cnt(dst_hbm, out_hbm, dia, ob, zb, acc, sem):
    cid = lax.axis_index("c")
    sid = lax.axis_index("s")
    w = _worker_id()

    def fill(i, _):
        def fcol(j, _):
            ob[i, pl.ds(j * 16, 16)] = jnp.ones((16,), jnp.float32)
            return 0
        return lax.fori_loop(0, D // 16, fcol, 0)

    lax.fori_loop(0, CC, fill, 0)

    def zfill(i, _):
        def zcol(j, _):
            zb[i, pl.ds(j * 16, 16)] = jnp.zeros((16,), jnp.float32)
            return 0
        return lax.fori_loop(0, D // 16, zcol, 0)

    lax.fori_loop(0, ZROWS, zfill, 0)

    def zstripe(k, _):
        pltpu.sync_copy(zb, acc.at[pl.ds(sid * SPT + k * ZROWS, ZROWS)])
        return 0

    lax.fori_loop(0, SPT // ZROWS, zstripe, 0)

    @pl.when(sid == 0)
    def _():
        def ztail(k, _):
            pltpu.sync_copy(zb, acc.at[pl.ds(NS * SPT + k * ZROWS, ZROWS)])
            return 0
        lax.fori_loop(0, TAIL // ZROWS, ztail, 0)

    pltpu.sync_copy(dst_hbm.at[pl.ds(w * EPW, EPW)], dia)
    plsc.subcore_barrier()

    def sca(c):
        return pltpu.make_async_copy(ob, acc.at[dia.at[pl.ds(c * CC, CC)]], sem)

    def group(gq, _):
        for j in range(5):
            sca(gq * 5 + j).start(add=True)
        for j in range(5):
            sca(gq * 5 + j).wait()
        return 0

    lax.fori_loop(0, NCHUNKC // 5, group, 0)
    plsc.subcore_barrier()

    pltpu.sync_copy(
        acc.at[pl.ds(sid * SPT, SPT)], out_hbm.at[cid, pl.ds(sid * SPT, SPT)]
    )

    @pl.when(sid == 0)
    def _():
        pltpu.sync_copy(
            acc.at[pl.ds(NS * SPT, TAIL)], out_hbm.at[cid, pl.ds(NS * SPT, TAIL)]
        )


_RB = 1000
_NB = N // _RB


def _tc_pre_body(h_ref, wl_ref, wr_ref, b_ref, y_ref, z_ref):
    h = h_ref[...]
    dn = (((1,), (1,)), ((), ()))
    y_ref[...] = lax.dot_general(h, wl_ref[...], dn, preferred_element_type=jnp.float32)
    z_ref[...] = (
        lax.dot_general(h, wr_ref[...], dn, preferred_element_type=jnp.float32)
        + b_ref[...]
    )


_tc_pre = pl.pallas_call(
    _tc_pre_body,
    grid=(_NB,),
    in_specs=[
        pl.BlockSpec((_RB, D), lambda i: (i, 0)),
        pl.BlockSpec((D, D), lambda i: (0, 0)),
        pl.BlockSpec((D, D), lambda i: (0, 0)),
        pl.BlockSpec((1, D), lambda i: (0, 0)),
    ],
    out_specs=[
        pl.BlockSpec((_RB, D), lambda i: (i, 0)),
        pl.BlockSpec((_RB, D), lambda i: (i, 0)),
    ],
    out_shape=[
        jax.ShapeDtypeStruct((N, D), jnp.float32),
        jax.ShapeDtypeStruct((N, D), jnp.float32),
    ],
)


def _tc_combo_body(p_ref, c_ref, z_ref, wl_ref, wr_ref, b_ref, y_ref, z2_ref):
    p = p_ref[0] + p_ref[1]
    cnt = (c_ref[0] + c_ref[1])[:, 0:1]
    inv = 1.0 / jnp.maximum(cnt, 1.0)
    h = jnp.maximum(p * inv + z_ref[...], 0.0)
    dn = (((1,), (1,)), ((), ()))
    y_ref[...] = lax.dot_general(h, wl_ref[...], dn, preferred_element_type=jnp.float32)
    z2_ref[...] = (
        lax.dot_general(h, wr_ref[...], dn, preferred_element_type=jnp.float32)
        + b_ref[...]
    )


_tc_combo = pl.pallas_call(
    _tc_combo_body,
    grid=(_NB,),
    in_specs=[
        pl.BlockSpec((NC, _RB, D), lambda i: (0, i, 0)),
        pl.BlockSpec((NC, _RB, D), lambda i: (0, i, 0)),
        pl.BlockSpec((_RB, D), lambda i: (i, 0)),
        pl.BlockSpec((D, D), lambda i: (0, 0)),
        pl.BlockSpec((D, D), lambda i: (0, 0)),
        pl.BlockSpec((1, D), lambda i: (0, 0)),
    ],
    out_specs=[
        pl.BlockSpec((_RB, D), lambda i: (i, 0)),
        pl.BlockSpec((_RB, D), lambda i: (i, 0)),
    ],
    out_shape=[
        jax.ShapeDtypeStruct((N, D), jnp.float32),
        jax.ShapeDtypeStruct((N, D), jnp.float32),
    ],
)


def _tc_post_body(relu, p_ref, c_ref, z_ref, o_ref):
    p = p_ref[0] + p_ref[1]
    cnt = (c_ref[0] + c_ref[1])[:, 0:1]
    inv = 1.0 / jnp.maximum(cnt, 1.0)
    out = p * inv + z_ref[...]
    if relu:
        out = jnp.maximum(out, 0.0)
    o_ref[...] = out


def _make_tc_post(relu):
    return pl.pallas_call(
        functools.partial(_tc_post_body, relu),
        grid=(_NB,),
        in_specs=[
            pl.BlockSpec((NC, _RB, D), lambda i: (0, i, 0)),
            pl.BlockSpec((NC, _RB, D), lambda i: (0, i, 0)),
            pl.BlockSpec((_RB, D), lambda i: (i, 0)),
        ],
        out_specs=pl.BlockSpec((_RB, D), lambda i: (i, 0)),
        out_shape=jax.ShapeDtypeStruct((N, D), jnp.float32),
    )


_tc_post_relu = _make_tc_post(True)
_tc_post_last = _make_tc_post(False)


def kernel(x, edge_index, Wl0, Wr0, b0, Wl1, Wr1, b1, Wl2, Wr2, b2):
    src = edge_index[0].astype(jnp.int32)
    dst = edge_index[1].astype(jnp.int32)
    cnt_parts = _sc_cnt(dst)
    y, z = _tc_pre(x, Wl0, Wr0, b0.reshape(1, D))
    for Wl, Wr, b in [(Wl1, Wr1, b1), (Wl2, Wr2, b2)]:
        parts = _sc_agg(y, src, dst)
        y, z = _tc_combo(parts, cnt_parts, z, Wl, Wr, b.reshape(1, D))
    parts = _sc_agg(y, src, dst)
    return _tc_post_last(parts, cnt_parts, z)


# speedup vs baseline: 11.9379x; 1.0092x over previous
import functools

import jax
import jax.numpy as jnp
from jax import lax
from jax.experimental import pallas as pl
from jax.experimental.pallas import tpu as pltpu
from jax.experimental.pallas import tpu_sc as plsc

N = 10000
E = 320000
D = 128

NC = 2
NS = 16
NW = NC * NS
EPW = E // NW
C = 80
NCHUNK = EPW // C
NB = 3
GL = 2
CC = 80
NCHUNKC = EPW // CC
SPT = 624
TAIL = N - NS * SPT
ZROWS = 8

_MESH = plsc.VectorSubcoreMesh(
    core_axis_name="c", subcore_axis_name="s", num_cores=NC, num_subcores=NS
)


def _worker_id():
    return lax.axis_index("c") * NS + lax.axis_index("s")


@functools.partial(
    pl.kernel,
    out_type=jax.ShapeDtypeStruct((NC, N, D), jnp.float32),
    mesh=_MESH,
    scratch_types=[
        pltpu.VMEM((EPW,), jnp.int32),
        pltpu.VMEM((EPW,), jnp.int32),
        [pltpu.VMEM((C, D), jnp.float32)] * NB,
        pltpu.VMEM_SHARED((N, D), jnp.float32),
        [pltpu.SemaphoreType.DMA] * NB,
        [pltpu.SemaphoreType.DMA] * NB,
    ],
)
def _sc_agg(y_hbm, src_hbm, dst_hbm, out_hbm, sia, dia, rbs, acc, sgs, sss):
    cid = lax.axis_index("c")
    sid = lax.axis_index("s")
    w = _worker_id()
    zb = rbs[0].at[pl.ds(0, ZROWS)]

    def zfill(i, _):
        def zcol(j, _):
            rbs[0][i, pl.ds(j * 16, 16)] = jnp.zeros((16,), jnp.float32)
            return 0
        return lax.fori_loop(0, D // 16, zcol, 0)

    lax.fori_loop(0, ZROWS, zfill, 0)

    def zstripe(k, _):
        pltpu.sync_copy(zb, acc.at[pl.ds(sid * SPT + k * ZROWS, ZROWS)])
        return 0

    lax.fori_loop(0, SPT // ZROWS, zstripe, 0)

    @pl.when(sid == 0)
    def _():
        def ztail(k, _):
            pltpu.sync_copy(zb, acc.at[pl.ds(NS * SPT + k * ZROWS, ZROWS)])
            return 0
        lax.fori_loop(0, TAIL // ZROWS, ztail, 0)

    pltpu.sync_copy(src_hbm.at[pl.ds(w * EPW, EPW)], sia)
    pltpu.sync_copy(dst_hbm.at[pl.ds(w * EPW, EPW)], dia)
    plsc.subcore_barrier()

    def gat(c, b):
        return pltpu.make_async_copy(
            y_hbm.at[sia.at[pl.ds(c * C, C)]], rbs[b], sgs[b]
        )

    def sca(c, b):
        return pltpu.make_async_copy(
            rbs[b], acc.at[dia.at[pl.ds(c * C, C)]], sss[b]
        )

    for c in range(GL):
        gat(c, c).start()

    STEPS = NCHUNK - GL
    MAIN = STEPS // NB
    REM = STEPS - MAIN * NB

    def group(g, _):
        for j in range(NB):
            c = NB * g + GL + j
            b = (GL + j) % NB

            @pl.when(c >= NB)
            def _():
                sca(c - NB, b).wait()

            gat(c, b).start()
            gat(c - GL, j).wait()
            sca(c - GL, j).start(add=True)
        return 0

    lax.fori_loop(0, MAIN, group, 0)

    for k in range(REM):
        c = MAIN * NB + GL + k
        sca(c - NB, c % NB).wait()
        gat(c, c % NB).start()
        gat(c - GL, (c - GL) % NB).wait()
        sca(c - GL, (c - GL) % NB).start(add=True)
    for c in range(NCHUNK - GL, NCHUNK):
        gat(c, c % NB).wait()
        sca(c, c % NB).start(add=True)
    for c in range(NCHUNK - NB, NCHUNK):
        sca(c, c % NB).wait()
    plsc.subcore_barrier()

    pltpu.sync_copy(
        acc.at[pl.ds(sid * SPT, SPT)], out_hbm.at[cid, pl.ds(sid * SPT, SPT)]
    )

    @pl.when(sid == 0)
    def _():
        pltpu.sync_copy(
            acc.at[pl.ds(NS * SPT, TAIL)], out_hbm.at[cid, pl.ds(NS * SPT, TAIL)]
        )


@functools.partial(
    pl.kernel,
    out_type=jax.ShapeDtypeStruct((NC, N, D), jnp.float32),
    mesh=_MESH,
    scratch_types=[
        pltpu.VMEM((EPW,), jnp.int32),
        pltpu.VMEM((CC, D), jnp.float32),
        pltpu.VMEM((ZROWS, D), jnp.float32),
        pltpu.VMEM_SHARED((N, D), jnp.float32),
        pltpu.SemaphoreType.DMA,
    ],
)
def _sc_cnt(dst_hbm, out_hbm, dia, ob, zb, acc, sem):
    cid = lax.axis_index("c")
    sid = lax.axis_index("s")
    w = _worker_id()

    def fill(i, _):
        def fcol(j, _):
            ob[i, pl.ds(j * 16, 16)] = jnp.ones((16,), jnp.float32)
            return 0
        return lax.fori_loop(0, D // 16, fcol, 0)

    lax.fori_loop(0, CC, fill, 0)

    def zfill(i, _):
        def zcol(j, _):
            zb[i, pl.ds(j * 16, 16)] = jnp.zeros((16,), jnp.float32)
            return 0
        return lax.fori_loop(0, D // 16, zcol, 0)

    lax.fori_loop(0, ZROWS, zfill, 0)

    def zstripe(k, _):
        pltpu.sync_copy(zb, acc.at[pl.ds(sid * SPT + k * ZROWS, ZROWS)])
        return 0

    lax.fori_loop(0, SPT // ZROWS, zstripe, 0)

    @pl.when(sid == 0)
    def _():
        def ztail(k, _):
            pltpu.sync_copy(zb, acc.at[pl.ds(NS * SPT + k * ZROWS, ZROWS)])
            return 0
        lax.fori_loop(0, TAIL // ZROWS, ztail, 0)

    pltpu.sync_copy(dst_hbm.at[pl.ds(w * EPW, EPW)], dia)
    plsc.subcore_barrier()

    def sca(c):
        return pltpu.make_async_copy(ob, acc.at[dia.at[pl.ds(c * CC, CC)]], sem)

    def group(gq, _):
        for j in range(5):
            sca(gq * 5 + j).start(add=True)
        for j in range(5):
            sca(gq * 5 + j).wait()
        return 0

    lax.fori_loop(0, NCHUNKC // 5, group, 0)
    plsc.subcore_barrier()

    pltpu.sync_copy(
        acc.at[pl.ds(sid * SPT, SPT)], out_hbm.at[cid, pl.ds(sid * SPT, SPT)]
    )

    @pl.when(sid == 0)
    def _():
        pltpu.sync_copy(
            acc.at[pl.ds(NS * SPT, TAIL)], out_hbm.at[cid, pl.ds(NS * SPT, TAIL)]
        )


_RB = 1000
_NB = N // _RB


def _tc_pre_body(h_ref, wl_ref, wr_ref, b_ref, y_ref, z_ref):
    h = h_ref[...]
    dn = (((1,), (1,)), ((), ()))
    y_ref[...] = lax.dot_general(h, wl_ref[...], dn, preferred_element_type=jnp.float32)
    z_ref[...] = (
        lax.dot_general(h, wr_ref[...], dn, preferred_element_type=jnp.float32)
        + b_ref[...]
    )


_tc_pre = pl.pallas_call(
    _tc_pre_body,
    grid=(_NB,),
    in_specs=[
        pl.BlockSpec((_RB, D), lambda i: (i, 0)),
        pl.BlockSpec((D, D), lambda i: (0, 0)),
        pl.BlockSpec((D, D), lambda i: (0, 0)),
        pl.BlockSpec((1, D), lambda i: (0, 0)),
    ],
    out_specs=[
        pl.BlockSpec((_RB, D), lambda i: (i, 0)),
        pl.BlockSpec((_RB, D), lambda i: (i, 0)),
    ],
    out_shape=[
        jax.ShapeDtypeStruct((N, D), jnp.float32),
        jax.ShapeDtypeStruct((N, D), jnp.float32),
    ],
)


def _tc_combo_body(p_ref, c_ref, z_ref, wl_ref, wr_ref, b_ref, y_ref, z2_ref):
    p = p_ref[0] + p_ref[1]
    cnt = (c_ref[0] + c_ref[1])[:, 0:1]
    inv = 1.0 / jnp.maximum(cnt, 1.0)
    h = jnp.maximum(p * inv + z_ref[...], 0.0)
    dn = (((1,), (1,)), ((), ()))
    y_ref[...] = lax.dot_general(h, wl_ref[...], dn, preferred_element_type=jnp.float32)
    z2_ref[...] = (
        lax.dot_general(h, wr_ref[...], dn, preferred_element_type=jnp.float32)
        + b_ref[...]
    )


_tc_combo = pl.pallas_call(
    _tc_combo_body,
    grid=(_NB,),
    in_specs=[
        pl.BlockSpec((NC, _RB, D), lambda i: (0, i, 0)),
        pl.BlockSpec((NC, _RB, D), lambda i: (0, i, 0)),
        pl.BlockSpec((_RB, D), lambda i: (i, 0)),
        pl.BlockSpec((D, D), lambda i: (0, 0)),
        pl.BlockSpec((D, D), lambda i: (0, 0)),
        pl.BlockSpec((1, D), lambda i: (0, 0)),
    ],
    out_specs=[
        pl.BlockSpec((_RB, D), lambda i: (i, 0)),
        pl.BlockSpec((_RB, D), lambda i: (i, 0)),
    ],
    out_shape=[
        jax.ShapeDtypeStruct((N, D), jnp.float32),
        jax.ShapeDtypeStruct((N, D), jnp.float32),
    ],
)


def _tc_post_body(relu, p_ref, c_ref, z_ref, o_ref):
    p = p_ref[0] + p_ref[1]
    cnt = (c_ref[0] + c_ref[1])[:, 0:1]
    inv = 1.0 / jnp.maximum(cnt, 1.0)
    out = p * inv + z_ref[...]
    if relu:
        out = jnp.maximum(out, 0.0)
    o_ref[...] = out


def _make_tc_post(relu):
    return pl.pallas_call(
        functools.partial(_tc_post_body, relu),
        grid=(_NB,),
        in_specs=[
            pl.BlockSpec((NC, _RB, D), lambda i: (0, i, 0)),
            pl.BlockSpec((NC, _RB, D), lambda i: (0, i, 0)),
            pl.BlockSpec((_RB, D), lambda i: (i, 0)),
        ],
        out_specs=pl.BlockSpec((_RB, D), lambda i: (i, 0)),
        out_shape=jax.ShapeDtypeStruct((N, D), jnp.float32),
    )


_tc_post_relu = _make_tc_post(True)
_tc_post_last = _make_tc_post(False)


def kernel(x, edge_index, Wl0, Wr0, b0, Wl1, Wr1, b1, Wl2, Wr2, b2):
    src = edge_index[0].astype(jnp.int32)
    dst = edge_index[1].astype(jnp.int32)
    cnt_parts = _sc_cnt(dst)
    y, z = _tc_pre(x, Wl0, Wr0, b0.reshape(1, D))
    for Wl, Wr, b in [(Wl1, Wr1, b1), (Wl2, Wr2, b2)]:
        parts = _sc_agg(y, src, dst)
        y, z = _tc_combo(parts, cnt_parts, z, Wl, Wr, b.reshape(1, D))
    parts = _sc_agg(y, src, dst)
    return _tc_post_last(parts, cnt_parts, z)
